# Initial kernel scaffold; baseline (speedup 1.0000x reference)
import jax
import jax.numpy as jnp
from jax.experimental import pallas as pl


def kernel(x, edge_index, W1, b1, W2, b2):
    raise NotImplementedError("write your pallas kernel here")


# speedup vs baseline: 14.4005x; 14.4005x over previous
import functools

import jax
import jax.numpy as jnp
from jax import lax
from jax.experimental import pallas as pl
from jax.experimental.pallas import tpu as pltpu
from jax.experimental.pallas import tpu_sc as plsc

N_NODES = 10000
D_IN = 128
EMB1 = 64
EMB2 = 16
E = 320000

NC, NS = 2, 16
NW = NC * NS
CHUNK = 128
K = 80
EPW = CHUNK * K
E_PAD = EPW * NW
N_PAD = 10240
ROWS_PER_SUB = N_PAD // NS
PAD_IDX = N_PAD - 1
DEG_W = 16

R_BLK = 1024

_mesh = plsc.VectorSubcoreMesh(core_axis_name="c", subcore_axis_name="s",
                               num_cores=NC, num_subcores=NS)
_sc_params = pltpu.CompilerParams(use_tc_tiling_on_sc=False)


@functools.partial(
    pl.kernel,
    out_type=jax.ShapeDtypeStruct((NC * N_PAD, DEG_W), jnp.float32),
    mesh=_mesh,
    scratch_types=[
        pltpu.VMEM((CHUNK,), jnp.int32),
        pltpu.VMEM((CHUNK, DEG_W), jnp.float32),
        pltpu.VMEM_SHARED((N_PAD, DEG_W), jnp.float32),
    ],
    compiler_params=_sc_params,
)
def _deg_kernel(dst_hbm, ones_hbm, zeros_hbm, out_hbm, idx_v, ones_v, deg_sh):
    c = lax.axis_index("c")
    s = lax.axis_index("s")
    w = c * NS + s
    pltpu.sync_copy(ones_hbm, ones_v)
    base = s * ROWS_PER_SUB
    pltpu.sync_copy(zeros_hbm, deg_sh.at[pl.ds(base, ROWS_PER_SUB)])
    plsc.subcore_barrier()

    @pl.loop(0, K)
    def _(j):
        pltpu.sync_copy(dst_hbm.at[pl.ds(w * EPW + j * CHUNK, CHUNK)], idx_v)
        pltpu.sync_copy(ones_v, deg_sh.at[idx_v], add=True)

    plsc.subcore_barrier()
    pltpu.sync_copy(deg_sh.at[pl.ds(base, ROWS_PER_SUB)],
                    out_hbm.at[pl.ds(c * N_PAD + base, ROWS_PER_SUB)])


def _make_agg(D):
    @functools.partial(
        pl.kernel,
        out_type=jax.ShapeDtypeStruct((NC * N_PAD, D), jnp.float32),
        mesh=_mesh,
        scratch_types=[
            pltpu.VMEM((CHUNK,), jnp.int32),
            pltpu.VMEM((CHUNK,), jnp.int32),
            pltpu.VMEM((CHUNK, D), jnp.float32),
            pltpu.VMEM_SHARED((N_PAD, D), jnp.float32),
            pltpu.SemaphoreType.DMA,
        ],
        compiler_params=_sc_params,
    )
    def agg(h_hbm, src_hbm, dst_hbm, zeros_hbm, out_hbm,
            src_v, dst_v, rows_v, agg_sh, sem):
        c = lax.axis_index("c")
        s = lax.axis_index("s")
        w = c * NS + s
        base = s * ROWS_PER_SUB
        pltpu.sync_copy(zeros_hbm, agg_sh.at[pl.ds(base, ROWS_PER_SUB)])
        plsc.subcore_barrier()

        @pl.loop(0, K)
        def _(j):
            e0 = w * EPW + j * CHUNK
            pltpu.sync_copy(src_hbm.at[pl.ds(e0, CHUNK)], src_v)
            pltpu.sync_copy(dst_hbm.at[pl.ds(e0, CHUNK)], dst_v)
            pltpu.async_copy(h_hbm.at[src_v], rows_v, sem).wait()
            pltpu.sync_copy(rows_v, agg_sh.at[dst_v], add=True)

        plsc.subcore_barrier()
        pltpu.sync_copy(agg_sh.at[pl.ds(base, ROWS_PER_SUB)],
                        out_hbm.at[pl.ds(c * N_PAD + base, ROWS_PER_SUB)])

    return agg


_agg64 = _make_agg(EMB1)
_agg16 = _make_agg(EMB2)


def _mm_body(x_ref, w_ref, o_ref):
    o_ref[...] = jnp.dot(x_ref[...], w_ref[...],
                         preferred_element_type=jnp.float32,
                         precision=lax.Precision.HIGHEST)


_h1_call = pl.pallas_call(
    _mm_body,
    grid=(N_PAD // R_BLK,),
    in_specs=[pl.BlockSpec((R_BLK, D_IN), lambda i: (i, 0)),
              pl.BlockSpec((D_IN, EMB1), lambda i: (0, 0))],
    out_specs=pl.BlockSpec((R_BLK, EMB1), lambda i: (i, 0)),
    out_shape=jax.ShapeDtypeStruct((N_PAD, EMB1), jnp.float32),
)


def _dis(dp_ref):
    deg = dp_ref[0, :] + dp_ref[1, :] + 1.0
    return lax.rsqrt(deg)


def _scale_body(dp_ref, h_ref, o_ref):
    o_ref[...] = h_ref[...] * _dis(dp_ref)[:, None]


_scale_call = pl.pallas_call(
    _scale_body,
    grid=(N_PAD // R_BLK,),
    in_specs=[pl.BlockSpec((NC, R_BLK), lambda i: (0, i)),
              pl.BlockSpec((R_BLK, EMB1), lambda i: (i, 0))],
    out_specs=pl.BlockSpec((R_BLK, EMB1), lambda i: (i, 0)),
    out_shape=jax.ShapeDtypeStruct((N_PAD, EMB1), jnp.float32),
)


def _layer1_body(p_ref, h_ref, dp_ref, b1_ref, w2_ref, o_ref):
    dis = _dis(dp_ref)
    agg = p_ref[0] + p_ref[1] + h_ref[...]
    z = jnp.maximum(agg * dis[:, None] + b1_ref[0, :][None, :], 0.0)
    o_ref[...] = jnp.dot(z * dis[:, None], w2_ref[...],
                         preferred_element_type=jnp.float32,
                         precision=lax.Precision.HIGHEST)


_layer1_call = pl.pallas_call(
    _layer1_body,
    grid=(N_PAD // R_BLK,),
    in_specs=[pl.BlockSpec((NC, R_BLK, EMB1), lambda i: (0, i, 0)),
              pl.BlockSpec((R_BLK, EMB1), lambda i: (i, 0)),
              pl.BlockSpec((NC, R_BLK), lambda i: (0, i)),
              pl.BlockSpec((1, EMB1), lambda i: (0, 0)),
              pl.BlockSpec((EMB1, EMB2), lambda i: (0, 0))],
    out_specs=pl.BlockSpec((R_BLK, EMB2), lambda i: (i, 0)),
    out_shape=jax.ShapeDtypeStruct((N_PAD, EMB2), jnp.float32),
)


def _layer2_body(q_ref, h2_ref, dp_ref, b2_ref, o_ref):
    dis = _dis(dp_ref)
    agg = q_ref[0] + q_ref[1] + h2_ref[...]
    o_ref[...] = jnp.maximum(agg * dis[:, None] + b2_ref[0, :][None, :], 0.0)


_layer2_call = pl.pallas_call(
    _layer2_body,
    grid=(N_PAD // R_BLK,),
    in_specs=[pl.BlockSpec((NC, R_BLK, EMB2), lambda i: (0, i, 0)),
              pl.BlockSpec((R_BLK, EMB2), lambda i: (i, 0)),
              pl.BlockSpec((NC, R_BLK), lambda i: (0, i)),
              pl.BlockSpec((1, EMB2), lambda i: (0, 0))],
    out_specs=pl.BlockSpec((R_BLK, EMB2), lambda i: (i, 0)),
    out_shape=jax.ShapeDtypeStruct((N_PAD, EMB2), jnp.float32),
)


def kernel(x, edge_index, W1, b1, W2, b2):
    x = x.astype(jnp.float32)
    src = edge_index[0].astype(jnp.int32)
    dst = edge_index[1].astype(jnp.int32)

    xp = jnp.zeros((N_PAD, D_IN), jnp.float32).at[:N_NODES].set(x)
    pad = jnp.full((E_PAD - E,), PAD_IDX, jnp.int32)
    src_f = jnp.concatenate([src, pad])
    dst_f = jnp.concatenate([dst, pad])

    ones_rows = jnp.ones((CHUNK, DEG_W), jnp.float32)
    zeros16 = jnp.zeros((ROWS_PER_SUB, DEG_W), jnp.float32)
    zeros64 = jnp.zeros((ROWS_PER_SUB, EMB1), jnp.float32)

    degp = _deg_kernel(dst_f, ones_rows, zeros16)
    dp = degp.reshape(NC, N_PAD, DEG_W)[:, :, 0]

    h1 = _h1_call(xp, W1)
    hs1 = _scale_call(dp, h1)
    p = _agg64(hs1, src_f, dst_f, zeros64).reshape(NC, N_PAD, EMB1)
    hs2 = _layer1_call(p, hs1, dp, b1.reshape(1, EMB1), W2)
    q = _agg16(hs2, src_f, dst_f, zeros16).reshape(NC, N_PAD, EMB2)
    out = _layer2_call(q, hs2, dp, b2.reshape(1, EMB2))
    return out[:N_NODES]


# speedup vs baseline: 21.0356x; 1.4608x over previous
import functools

import jax
import jax.numpy as jnp
from jax import lax
from jax.experimental import pallas as pl
from jax.experimental.pallas import tpu as pltpu
from jax.experimental.pallas import tpu_sc as plsc

N_NODES = 10000
D_IN = 128
EMB1 = 64
EMB2 = 16
E = 320000

NC, NS = 2, 16
NW = NC * NS
CHUNK = 128
K = 80
EPW = CHUNK * K
E_PAD = EPW * NW
N_PAD = 10240
ROWS_PER_SUB = N_PAD // NS
PAD_IDX = N_PAD - 1
DEG_W = 16
GROUP = 8
NGRP = K // GROUP

R_BLK = 1024

_mesh = plsc.VectorSubcoreMesh(core_axis_name="c", subcore_axis_name="s",
                               num_cores=NC, num_subcores=NS)
_sc_params = pltpu.CompilerParams(use_tc_tiling_on_sc=False)


@functools.partial(
    pl.kernel,
    out_type=jax.ShapeDtypeStruct((NC * N_PAD, DEG_W), jnp.float32),
    mesh=_mesh,
    scratch_types=[
        *[pltpu.VMEM((CHUNK,), jnp.int32) for _ in range(GROUP)],
        pltpu.VMEM((CHUNK, DEG_W), jnp.float32),
        pltpu.VMEM_SHARED((N_PAD, DEG_W), jnp.float32),
        pltpu.SemaphoreType.DMA,
        pltpu.SemaphoreType.DMA,
    ],
    compiler_params=_sc_params,
)
def _deg_kernel(dst_hbm, ones_hbm, zeros_hbm, out_hbm,
                d0, d1, d2, d3, d4, d5, d6, d7, ones_v, deg_sh, isem, ssem):
    c = lax.axis_index("c")
    s = lax.axis_index("s")
    w = c * NS + s
    didx = [d0, d1, d2, d3, d4, d5, d6, d7]
    pltpu.sync_copy(ones_hbm, ones_v)
    base = s * ROWS_PER_SUB
    pltpu.sync_copy(zeros_hbm, deg_sh.at[pl.ds(base, ROWS_PER_SUB)])
    plsc.subcore_barrier()

    @pl.loop(0, NGRP)
    def _(g):
        e0 = w * EPW + g * GROUP * CHUNK
        loads = [
            pltpu.async_copy(dst_hbm.at[pl.ds(e0 + b * CHUNK, CHUNK)],
                             didx[b], isem)
            for b in range(GROUP)
        ]
        for dsc in loads:
            dsc.wait()
        for b in range(GROUP):
            pltpu.async_copy(ones_v, deg_sh.at[didx[b]], ssem,
                             add=True).wait()

    plsc.subcore_barrier()
    pltpu.sync_copy(deg_sh.at[pl.ds(base, ROWS_PER_SUB)],
                    out_hbm.at[pl.ds(c * N_PAD + base, ROWS_PER_SUB)])


def _make_agg(D):
    @functools.partial(
        pl.kernel,
        out_type=jax.ShapeDtypeStruct((NC * N_PAD, D), jnp.float32),
        mesh=_mesh,
        scratch_types=[
            pltpu.VMEM((K, CHUNK), jnp.int32),
            *[pltpu.VMEM((CHUNK,), jnp.int32) for _ in range(GROUP)],
            pltpu.VMEM((GROUP, CHUNK, D), jnp.float32),
            pltpu.VMEM_SHARED((N_PAD, D), jnp.float32),
            pltpu.SemaphoreType.DMA,
            pltpu.SemaphoreType.DMA,
            pltpu.SemaphoreType.DMA,
        ],
        compiler_params=_sc_params,
    )
    def agg(h_hbm, src_hbm, dst_hbm, zeros_hbm, out_hbm,
            src_all, d0, d1, d2, d3, d4, d5, d6, d7,
            rows_v, agg_sh, isem, gsem, ssem):
        c = lax.axis_index("c")
        s = lax.axis_index("s")
        w = c * NS + s
        didx = [d0, d1, d2, d3, d4, d5, d6, d7]
        base = s * ROWS_PER_SUB
        pltpu.sync_copy(src_hbm.at[w], src_all)
        pltpu.sync_copy(zeros_hbm, agg_sh.at[pl.ds(base, ROWS_PER_SUB)])
        plsc.subcore_barrier()

        @pl.loop(0, NGRP)
        def _(g):
            j0 = g * GROUP
            e0 = w * EPW + j0 * CHUNK
            loads = [
                pltpu.async_copy(dst_hbm.at[pl.ds(e0 + b * CHUNK, CHUNK)],
                                 didx[b], isem)
                for b in range(GROUP)
            ]
            gathers = [
                pltpu.async_copy(h_hbm.at[src_all.at[j0 + b]], rows_v.at[b], gsem)
                for b in range(GROUP)
            ]
            for dsc in loads:
                dsc.wait()
            for dsc in gathers:
                dsc.wait()
            for b in range(GROUP):
                pltpu.async_copy(rows_v.at[b], agg_sh.at[didx[b]], ssem,
                                 add=True).wait()

        plsc.subcore_barrier()
        pltpu.sync_copy(agg_sh.at[pl.ds(base, ROWS_PER_SUB)],
                        out_hbm.at[pl.ds(c * N_PAD + base, ROWS_PER_SUB)])

    return agg


_agg64 = _make_agg(EMB1)
_agg16 = _make_agg(EMB2)


def _mm_body(x_ref, w_ref, o_ref):
    o_ref[...] = jnp.dot(x_ref[...], w_ref[...],
                         preferred_element_type=jnp.float32,
                         precision=lax.Precision.HIGHEST)


_h1_call = pl.pallas_call(
    _mm_body,
    grid=(N_PAD // R_BLK,),
    in_specs=[pl.BlockSpec((R_BLK, D_IN), lambda i: (i, 0)),
              pl.BlockSpec((D_IN, EMB1), lambda i: (0, 0))],
    out_specs=pl.BlockSpec((R_BLK, EMB1), lambda i: (i, 0)),
    out_shape=jax.ShapeDtypeStruct((N_PAD, EMB1), jnp.float32),
)


def _dis(dp_ref):
    deg = dp_ref[0, :] + dp_ref[1, :] + 1.0
    return lax.rsqrt(deg)


def _scale_body(dp_ref, h_ref, o_ref):
    o_ref[...] = h_ref[...] * _dis(dp_ref)[:, None]


_scale_call = pl.pallas_call(
    _scale_body,
    grid=(N_PAD // R_BLK,),
    in_specs=[pl.BlockSpec((NC, R_BLK), lambda i: (0, i)),
              pl.BlockSpec((R_BLK, EMB1), lambda i: (i, 0))],
    out_specs=pl.BlockSpec((R_BLK, EMB1), lambda i: (i, 0)),
    out_shape=jax.ShapeDtypeStruct((N_PAD, EMB1), jnp.float32),
)


def _layer1_body(p_ref, h_ref, dp_ref, b1_ref, w2_ref, o_ref):
    dis = _dis(dp_ref)
    agg = p_ref[0] + p_ref[1] + h_ref[...]
    z = jnp.maximum(agg * dis[:, None] + b1_ref[0, :][None, :], 0.0)
    o_ref[...] = jnp.dot(z * dis[:, None], w2_ref[...],
                         preferred_element_type=jnp.float32,
                         precision=lax.Precision.HIGHEST)


_layer1_call = pl.pallas_call(
    _layer1_body,
    grid=(N_PAD // R_BLK,),
    in_specs=[pl.BlockSpec((NC, R_BLK, EMB1), lambda i: (0, i, 0)),
              pl.BlockSpec((R_BLK, EMB1), lambda i: (i, 0)),
              pl.BlockSpec((NC, R_BLK), lambda i: (0, i)),
              pl.BlockSpec((1, EMB1), lambda i: (0, 0)),
              pl.BlockSpec((EMB1, EMB2), lambda i: (0, 0))],
    out_specs=pl.BlockSpec((R_BLK, EMB2), lambda i: (i, 0)),
    out_shape=jax.ShapeDtypeStruct((N_PAD, EMB2), jnp.float32),
)


def _layer2_body(q_ref, h2_ref, dp_ref, b2_ref, o_ref):
    dis = _dis(dp_ref)
    agg = q_ref[0] + q_ref[1] + h2_ref[...]
    o_ref[...] = jnp.maximum(agg * dis[:, None] + b2_ref[0, :][None, :], 0.0)


_layer2_call = pl.pallas_call(
    _layer2_body,
    grid=(N_PAD // R_BLK,),
    in_specs=[pl.BlockSpec((NC, R_BLK, EMB2), lambda i: (0, i, 0)),
              pl.BlockSpec((R_BLK, EMB2), lambda i: (i, 0)),
              pl.BlockSpec((NC, R_BLK), lambda i: (0, i)),
              pl.BlockSpec((1, EMB2), lambda i: (0, 0))],
    out_specs=pl.BlockSpec((R_BLK, EMB2), lambda i: (i, 0)),
    out_shape=jax.ShapeDtypeStruct((N_PAD, EMB2), jnp.float32),
)


def kernel(x, edge_index, W1, b1, W2, b2):
    x = x.astype(jnp.float32)
    src = edge_index[0].astype(jnp.int32)
    dst = edge_index[1].astype(jnp.int32)

    xp = jnp.zeros((N_PAD, D_IN), jnp.float32).at[:N_NODES].set(x)
    pad = jnp.full((E_PAD - E,), PAD_IDX, jnp.int32)
    src_r = jnp.concatenate([src, pad]).reshape(NW, K, CHUNK)
    dst_f = jnp.concatenate([dst, pad])

    ones_rows = jnp.ones((CHUNK, DEG_W), jnp.float32)
    zeros16 = jnp.zeros((ROWS_PER_SUB, DEG_W), jnp.float32)
    zeros64 = jnp.zeros((ROWS_PER_SUB, EMB1), jnp.float32)

    degp = _deg_kernel(dst_f, ones_rows, zeros16)
    dp = degp.reshape(NC, N_PAD, DEG_W)[:, :, 0]

    h1 = _h1_call(xp, W1)
    hs1 = _scale_call(dp, h1)
    p = _agg64(hs1, src_r, dst_f, zeros64).reshape(NC, N_PAD, EMB1)
    hs2 = _layer1_call(p, hs1, dp, b1.reshape(1, EMB1), W2)
    q = _agg16(hs2, src_r, dst_f, zeros16).reshape(NC, N_PAD, EMB2)
    out = _layer2_call(q, hs2, dp, b2.reshape(1, EMB2))
    return out[:N_NODES]


# speedup vs baseline: 21.0661x; 1.0014x over previous
import functools

import jax
import jax.numpy as jnp
from jax import lax
from jax.experimental import pallas as pl
from jax.experimental.pallas import tpu as pltpu
from jax.experimental.pallas import tpu_sc as plsc

N_NODES = 10000
D_IN = 128
EMB1 = 64
EMB2 = 16
E = 320000

NC, NS = 2, 16
NW = NC * NS
CHUNK = 128
K = 80
EPW = CHUNK * K
E_PAD = EPW * NW
N_PAD = 10240
ROWS_PER_SUB = N_PAD // NS
PAD_IDX = N_PAD - 1
DEG_W = 16
GROUP = 8
NGRP = K // GROUP

R_BLK = 1024

_mesh = plsc.VectorSubcoreMesh(core_axis_name="c", subcore_axis_name="s",
                               num_cores=NC, num_subcores=NS)
_sc_params = pltpu.CompilerParams(use_tc_tiling_on_sc=False)


@functools.partial(
    pl.kernel,
    out_type=jax.ShapeDtypeStruct((NC * N_PAD, DEG_W), jnp.float32),
    mesh=_mesh,
    scratch_types=[
        *[pltpu.VMEM((CHUNK,), jnp.int32) for _ in range(GROUP)],
        pltpu.VMEM((CHUNK, DEG_W), jnp.float32),
        pltpu.VMEM_SHARED((N_PAD, DEG_W), jnp.float32),
        pltpu.SemaphoreType.DMA,
        pltpu.SemaphoreType.DMA,
    ],
    compiler_params=_sc_params,
)
def _deg_kernel(dst_hbm, ones_hbm, zeros_hbm, out_hbm,
                d0, d1, d2, d3, d4, d5, d6, d7, ones_v, deg_sh, isem, ssem):
    c = lax.axis_index("c")
    s = lax.axis_index("s")
    w = c * NS + s
    didx = [d0, d1, d2, d3, d4, d5, d6, d7]
    pltpu.sync_copy(ones_hbm, ones_v)
    base = s * ROWS_PER_SUB
    pltpu.sync_copy(zeros_hbm, deg_sh.at[pl.ds(base, ROWS_PER_SUB)])
    plsc.subcore_barrier()

    @pl.loop(0, NGRP)
    def _(g):
        e0 = w * EPW + g * GROUP * CHUNK
        loads = [
            pltpu.async_copy(dst_hbm.at[pl.ds(e0 + b * CHUNK, CHUNK)],
                             didx[b], isem)
            for b in range(GROUP)
        ]
        for dsc in loads:
            dsc.wait()
        for b in range(GROUP):
            pltpu.async_copy(ones_v, deg_sh.at[didx[b]], ssem,
                             add=True).wait()

    plsc.subcore_barrier()
    pltpu.sync_copy(deg_sh.at[pl.ds(base, ROWS_PER_SUB)],
                    out_hbm.at[pl.ds(c * N_PAD + base, ROWS_PER_SUB)])


def _make_agg(D):
    @functools.partial(
        pl.kernel,
        out_type=jax.ShapeDtypeStruct((NC * N_PAD, D), jnp.float32),
        mesh=_mesh,
        scratch_types=[
            pltpu.VMEM((K, CHUNK), jnp.int32),
            *[pltpu.VMEM((CHUNK,), jnp.int32) for _ in range(GROUP)],
            pltpu.VMEM((GROUP, CHUNK, D), jnp.float32),
            pltpu.VMEM_SHARED((N_PAD, D), jnp.float32),
            pltpu.SemaphoreType.DMA,
            pltpu.SemaphoreType.DMA,
            pltpu.SemaphoreType.DMA,
            pltpu.SemaphoreType.DMA,
        ],
        compiler_params=_sc_params,
    )
    def agg(h_hbm, src_hbm, dst_hbm, zeros_hbm, out_hbm,
            src_all, d0, d1, d2, d3, d4, d5, d6, d7,
            rows_v, agg_sh, isem, gsem, ssem, ssem2):
        c = lax.axis_index("c")
        s = lax.axis_index("s")
        w = c * NS + s
        didx = [d0, d1, d2, d3, d4, d5, d6, d7]
        base = s * ROWS_PER_SUB
        pltpu.sync_copy(src_hbm.at[w], src_all)
        pltpu.sync_copy(zeros_hbm, agg_sh.at[pl.ds(base, ROWS_PER_SUB)])
        plsc.subcore_barrier()

        @pl.loop(0, NGRP)
        def _(g):
            j0 = g * GROUP
            e0 = w * EPW + j0 * CHUNK
            loads = [
                pltpu.async_copy(dst_hbm.at[pl.ds(e0 + b * CHUNK, CHUNK)],
                                 didx[b], isem)
                for b in range(GROUP)
            ]
            gathers = [
                pltpu.async_copy(h_hbm.at[src_all.at[j0 + b]], rows_v.at[b], gsem)
                for b in range(GROUP)
            ]
            for dsc in loads:
                dsc.wait()
            for dsc in gathers:
                dsc.wait()
            for b in range(0, GROUP, 2):
                a0 = pltpu.async_copy(rows_v.at[b], agg_sh.at[didx[b]], ssem,
                                      add=True)
                a1 = pltpu.async_copy(rows_v.at[b + 1], agg_sh.at[didx[b + 1]],
                                      ssem2, add=True)
                a0.wait()
                a1.wait()

        plsc.subcore_barrier()
        pltpu.sync_copy(agg_sh.at[pl.ds(base, ROWS_PER_SUB)],
                        out_hbm.at[pl.ds(c * N_PAD + base, ROWS_PER_SUB)])

    return agg


_agg64 = _make_agg(EMB1)
_agg16 = _make_agg(EMB2)


def _mm_body(x_ref, w_ref, o_ref):
    o_ref[...] = jnp.dot(x_ref[...], w_ref[...],
                         preferred_element_type=jnp.float32,
                         precision=lax.Precision.HIGHEST)


_h1_call = pl.pallas_call(
    _mm_body,
    grid=(N_PAD // R_BLK,),
    in_specs=[pl.BlockSpec((R_BLK, D_IN), lambda i: (i, 0)),
              pl.BlockSpec((D_IN, EMB1), lambda i: (0, 0))],
    out_specs=pl.BlockSpec((R_BLK, EMB1), lambda i: (i, 0)),
    out_shape=jax.ShapeDtypeStruct((N_PAD, EMB1), jnp.float32),
)


def _dis(dp_ref):
    deg = dp_ref[0, :] + dp_ref[1, :] + 1.0
    return lax.rsqrt(deg)


def _scale_body(dp_ref, h_ref, o_ref):
    o_ref[...] = h_ref[...] * _dis(dp_ref)[:, None]


_scale_call = pl.pallas_call(
    _scale_body,
    grid=(N_PAD // R_BLK,),
    in_specs=[pl.BlockSpec((NC, R_BLK), lambda i: (0, i)),
              pl.BlockSpec((R_BLK, EMB1), lambda i: (i, 0))],
    out_specs=pl.BlockSpec((R_BLK, EMB1), lambda i: (i, 0)),
    out_shape=jax.ShapeDtypeStruct((N_PAD, EMB1), jnp.float32),
)


def _layer1_body(p_ref, h_ref, dp_ref, b1_ref, w2_ref, o_ref):
    dis = _dis(dp_ref)
    agg = p_ref[0] + p_ref[1] + h_ref[...]
    z = jnp.maximum(agg * dis[:, None] + b1_ref[0, :][None, :], 0.0)
    o_ref[...] = jnp.dot(z * dis[:, None], w2_ref[...],
                         preferred_element_type=jnp.float32,
                         precision=lax.Precision.HIGHEST)


_layer1_call = pl.pallas_call(
    _layer1_body,
    grid=(N_PAD // R_BLK,),
    in_specs=[pl.BlockSpec((NC, R_BLK, EMB1), lambda i: (0, i, 0)),
              pl.BlockSpec((R_BLK, EMB1), lambda i: (i, 0)),
              pl.BlockSpec((NC, R_BLK), lambda i: (0, i)),
              pl.BlockSpec((1, EMB1), lambda i: (0, 0)),
              pl.BlockSpec((EMB1, EMB2), lambda i: (0, 0))],
    out_specs=pl.BlockSpec((R_BLK, EMB2), lambda i: (i, 0)),
    out_shape=jax.ShapeDtypeStruct((N_PAD, EMB2), jnp.float32),
)


def _layer2_body(q_ref, h2_ref, dp_ref, b2_ref, o_ref):
    dis = _dis(dp_ref)
    agg = q_ref[0] + q_ref[1] + h2_ref[...]
    o_ref[...] = jnp.maximum(agg * dis[:, None] + b2_ref[0, :][None, :], 0.0)


_layer2_call = pl.pallas_call(
    _layer2_body,
    grid=(N_PAD // R_BLK,),
    in_specs=[pl.BlockSpec((NC, R_BLK, EMB2), lambda i: (0, i, 0)),
              pl.BlockSpec((R_BLK, EMB2), lambda i: (i, 0)),
              pl.BlockSpec((NC, R_BLK), lambda i: (0, i)),
              pl.BlockSpec((1, EMB2), lambda i: (0, 0))],
    out_specs=pl.BlockSpec((R_BLK, EMB2), lambda i: (i, 0)),
    out_shape=jax.ShapeDtypeStruct((N_PAD, EMB2), jnp.float32),
)


def kernel(x, edge_index, W1, b1, W2, b2):
    x = x.astype(jnp.float32)
    src = edge_index[0].astype(jnp.int32)
    dst = edge_index[1].astype(jnp.int32)

    xp = jnp.zeros((N_PAD, D_IN), jnp.float32).at[:N_NODES].set(x)
    pad = jnp.full((E_PAD - E,), PAD_IDX, jnp.int32)
    src_r = jnp.concatenate([src, pad]).reshape(NW, K, CHUNK)
    dst_f = jnp.concatenate([dst, pad])

    ones_rows = jnp.ones((CHUNK, DEG_W), jnp.float32)
    zeros16 = jnp.zeros((ROWS_PER_SUB, DEG_W), jnp.float32)
    zeros64 = jnp.zeros((ROWS_PER_SUB, EMB1), jnp.float32)

    degp = _deg_kernel(dst_f, ones_rows, zeros16)
    dp = degp.reshape(NC, N_PAD, DEG_W)[:, :, 0]

    h1 = _h1_call(xp, W1)
    hs1 = _scale_call(dp, h1)
    p = _agg64(hs1, src_r, dst_f, zeros64).reshape(NC, N_PAD, EMB1)
    hs2 = _layer1_call(p, hs1, dp, b1.reshape(1, EMB1), W2)
    q = _agg16(hs2, src_r, dst_f, zeros16).reshape(NC, N_PAD, EMB2)
    out = _layer2_call(q, hs2, dp, b2.reshape(1, EMB2))
    return out[:N_NODES]


# speedup vs baseline: 21.9891x; 1.0438x over previous
import functools

import jax
import jax.numpy as jnp
from jax import lax
from jax.experimental import pallas as pl
from jax.experimental.pallas import tpu as pltpu
from jax.experimental.pallas import tpu_sc as plsc

N_NODES = 10000
D_IN = 128
EMB1 = 64
EMB2 = 16
E = 320000

NC, NS = 2, 16
NW = NC * NS
CHUNK = 128
K = 80
EPW = CHUNK * K
E_PAD = EPW * NW
N_PAD = 10240
ROWS_PER_SUB = N_PAD // NS
PAD_IDX = N_PAD - 1
DEG_W = 16
GROUP = 8
NGRP = K // GROUP
AGRP = 4
ANGRP = K // AGRP

R_BLK = 1024

_mesh = plsc.VectorSubcoreMesh(core_axis_name="c", subcore_axis_name="s",
                               num_cores=NC, num_subcores=NS)
_sc_params = pltpu.CompilerParams(use_tc_tiling_on_sc=False)


@functools.partial(
    pl.kernel,
    out_type=jax.ShapeDtypeStruct((NC * N_PAD, DEG_W), jnp.float32),
    mesh=_mesh,
    scratch_types=[
        *[pltpu.VMEM((CHUNK,), jnp.int32) for _ in range(GROUP)],
        pltpu.VMEM((CHUNK, DEG_W), jnp.float32),
        pltpu.VMEM_SHARED((N_PAD, DEG_W), jnp.float32),
        pltpu.SemaphoreType.DMA,
        pltpu.SemaphoreType.DMA,
    ],
    compiler_params=_sc_params,
)
def _deg_kernel(dst_hbm, ones_hbm, zeros_hbm, out_hbm,
                d0, d1, d2, d3, d4, d5, d6, d7, ones_v, deg_sh, isem, ssem):
    c = lax.axis_index("c")
    s = lax.axis_index("s")
    w = c * NS + s
    didx = [d0, d1, d2, d3, d4, d5, d6, d7]
    pltpu.sync_copy(ones_hbm, ones_v)
    base = s * ROWS_PER_SUB
    pltpu.sync_copy(zeros_hbm, deg_sh.at[pl.ds(base, ROWS_PER_SUB)])
    plsc.subcore_barrier()

    @pl.loop(0, NGRP)
    def _(g):
        e0 = w * EPW + g * GROUP * CHUNK
        loads = [
            pltpu.async_copy(dst_hbm.at[pl.ds(e0 + b * CHUNK, CHUNK)],
                             didx[b], isem)
            for b in range(GROUP)
        ]
        for dsc in loads:
            dsc.wait()
        for b in range(GROUP):
            pltpu.async_copy(ones_v, deg_sh.at[didx[b]], ssem,
                             add=True).wait()

    plsc.subcore_barrier()
    pltpu.sync_copy(deg_sh.at[pl.ds(base, ROWS_PER_SUB)],
                    out_hbm.at[pl.ds(c * N_PAD + base, ROWS_PER_SUB)])


def _make_agg(D):
    @functools.partial(
        pl.kernel,
        out_type=jax.ShapeDtypeStruct((NC * N_PAD, D), jnp.float32),
        mesh=_mesh,
        scratch_types=[
            pltpu.VMEM((K, CHUNK), jnp.int32),
            *[pltpu.VMEM((CHUNK,), jnp.int32) for _ in range(2 * AGRP)],
            pltpu.VMEM((AGRP, CHUNK, D), jnp.float32),
            pltpu.VMEM((AGRP, CHUNK, D), jnp.float32),
            pltpu.VMEM_SHARED((N_PAD, D), jnp.float32),
            pltpu.SemaphoreType.DMA,
            pltpu.SemaphoreType.DMA,
            pltpu.SemaphoreType.DMA,
            pltpu.SemaphoreType.DMA,
            pltpu.SemaphoreType.DMA,
            pltpu.SemaphoreType.DMA,
        ],
        compiler_params=_sc_params,
    )
    def agg(h_hbm, src_hbm, dst_hbm, zeros_hbm, out_hbm,
            src_all, d0, d1, d2, d3, d4, d5, d6, d7,
            rows_a, rows_b, agg_sh, gsa, gsb, ssa, ssb, isa, isb):
        c = lax.axis_index("c")
        s = lax.axis_index("s")
        w = c * NS + s
        banks = [[d0, d1, d2, d3], [d4, d5, d6, d7]]
        rows = [rows_a, rows_b]
        gsems = [gsa, gsb]
        ssems = [ssa, ssb]
        isems = [isa, isb]
        base = s * ROWS_PER_SUB
        pltpu.sync_copy(src_hbm.at[w], src_all)
        pltpu.sync_copy(zeros_hbm, agg_sh.at[pl.ds(base, ROWS_PER_SUB)])
        plsc.subcore_barrier()

        def gathers(g, p):
            return [
                pltpu.async_copy(h_hbm.at[src_all.at[g * AGRP + b]],
                                 rows[p].at[b], gsems[p])
                for b in range(AGRP)
            ]

        def loads(g, p):
            e0 = w * EPW + g * AGRP * CHUNK
            return [
                pltpu.async_copy(dst_hbm.at[pl.ds(e0 + b * CHUNK, CHUNK)],
                                 banks[p][b], isems[p])
                for b in range(AGRP)
            ]

        pend_g = gathers(0, 0)
        pend_i = loads(0, 0)
        pend_s = None
        for g in range(ANGRP):
            p = g % 2
            for dsc in pend_g:
                dsc.wait()
            for dsc in pend_i:
                dsc.wait()
            if pend_s is not None:
                for dsc in pend_s:
                    dsc.wait()
            if g + 1 < ANGRP:
                pend_g = gathers(g + 1, 1 - p)
                pend_i = loads(g + 1, 1 - p)
            pend_s = [
                pltpu.async_copy(rows[p].at[b], agg_sh.at[banks[p][b]],
                                 ssems[p], add=True)
                for b in range(AGRP)
            ]
        for dsc in pend_s:
            dsc.wait()

        plsc.subcore_barrier()
        pltpu.sync_copy(agg_sh.at[pl.ds(base, ROWS_PER_SUB)],
                        out_hbm.at[pl.ds(c * N_PAD + base, ROWS_PER_SUB)])

    return agg


_agg64 = _make_agg(EMB1)
_agg16 = _make_agg(EMB2)


def _mm_body(x_ref, w_ref, o_ref):
    o_ref[...] = jnp.dot(x_ref[...], w_ref[...],
                         preferred_element_type=jnp.float32,
                         precision=lax.Precision.HIGHEST)


_h1_call = pl.pallas_call(
    _mm_body,
    grid=(N_PAD // R_BLK,),
    in_specs=[pl.BlockSpec((R_BLK, D_IN), lambda i: (i, 0)),
              pl.BlockSpec((D_IN, EMB1), lambda i: (0, 0))],
    out_specs=pl.BlockSpec((R_BLK, EMB1), lambda i: (i, 0)),
    out_shape=jax.ShapeDtypeStruct((N_PAD, EMB1), jnp.float32),
)


def _dis(dp_ref):
    deg = dp_ref[0, :] + dp_ref[1, :] + 1.0
    return lax.rsqrt(deg)


def _scale_body(dp_ref, h_ref, o_ref):
    o_ref[...] = h_ref[...] * _dis(dp_ref)[:, None]


_scale_call = pl.pallas_call(
    _scale_body,
    grid=(N_PAD // R_BLK,),
    in_specs=[pl.BlockSpec((NC, R_BLK), lambda i: (0, i)),
              pl.BlockSpec((R_BLK, EMB1), lambda i: (i, 0))],
    out_specs=pl.BlockSpec((R_BLK, EMB1), lambda i: (i, 0)),
    out_shape=jax.ShapeDtypeStruct((N_PAD, EMB1), jnp.float32),
)


def _layer1_body(p_ref, h_ref, dp_ref, b1_ref, w2_ref, o_ref):
    dis = _dis(dp_ref)
    agg = p_ref[0] + p_ref[1] + h_ref[...]
    z = jnp.maximum(agg * dis[:, None] + b1_ref[0, :][None, :], 0.0)
    o_ref[...] = jnp.dot(z * dis[:, None], w2_ref[...],
                         preferred_element_type=jnp.float32,
                         precision=lax.Precision.HIGHEST)


_layer1_call = pl.pallas_call(
    _layer1_body,
    grid=(N_PAD // R_BLK,),
    in_specs=[pl.BlockSpec((NC, R_BLK, EMB1), lambda i: (0, i, 0)),
              pl.BlockSpec((R_BLK, EMB1), lambda i: (i, 0)),
              pl.BlockSpec((NC, R_BLK), lambda i: (0, i)),
              pl.BlockSpec((1, EMB1), lambda i: (0, 0)),
              pl.BlockSpec((EMB1, EMB2), lambda i: (0, 0))],
    out_specs=pl.BlockSpec((R_BLK, EMB2), lambda i: (i, 0)),
    out_shape=jax.ShapeDtypeStruct((N_PAD, EMB2), jnp.float32),
)


def _layer2_body(q_ref, h2_ref, dp_ref, b2_ref, o_ref):
    dis = _dis(dp_ref)
    agg = q_ref[0] + q_ref[1] + h2_ref[...]
    o_ref[...] = jnp.maximum(agg * dis[:, None] + b2_ref[0, :][None, :], 0.0)


_layer2_call = pl.pallas_call(
    _layer2_body,
    grid=(N_PAD // R_BLK,),
    in_specs=[pl.BlockSpec((NC, R_BLK, EMB2), lambda i: (0, i, 0)),
              pl.BlockSpec((R_BLK, EMB2), lambda i: (i, 0)),
              pl.BlockSpec((NC, R_BLK), lambda i: (0, i)),
              pl.BlockSpec((1, EMB2), lambda i: (0, 0))],
    out_specs=pl.BlockSpec((R_BLK, EMB2), lambda i: (i, 0)),
    out_shape=jax.ShapeDtypeStruct((N_PAD, EMB2), jnp.float32),
)


def kernel(x, edge_index, W1, b1, W2, b2):
    x = x.astype(jnp.float32)
    src = edge_index[0].astype(jnp.int32)
    dst = edge_index[1].astype(jnp.int32)

    xp = jnp.zeros((N_PAD, D_IN), jnp.float32).at[:N_NODES].set(x)
    pad = jnp.full((E_PAD - E,), PAD_IDX, jnp.int32)
    src_r = jnp.concatenate([src, pad]).reshape(NW, K, CHUNK)
    dst_f = jnp.concatenate([dst, pad])

    ones_rows = jnp.ones((CHUNK, DEG_W), jnp.float32)
    zeros16 = jnp.zeros((ROWS_PER_SUB, DEG_W), jnp.float32)
    zeros64 = jnp.zeros((ROWS_PER_SUB, EMB1), jnp.float32)

    degp = _deg_kernel(dst_f, ones_rows, zeros16)
    dp = degp.reshape(NC, N_PAD, DEG_W)[:, :, 0]

    h1 = _h1_call(xp, W1)
    hs1 = _scale_call(dp, h1)
    p = _agg64(hs1, src_r, dst_f, zeros64).reshape(NC, N_PAD, EMB1)
    hs2 = _layer1_call(p, hs1, dp, b1.reshape(1, EMB1), W2)
    q = _agg16(hs2, src_r, dst_f, zeros16).reshape(NC, N_PAD, EMB2)
    out = _layer2_call(q, hs2, dp, b2.reshape(1, EMB2))
    return out[:N_NODES]


# speedup vs baseline: 38.5287x; 1.7522x over previous
import functools

import jax
import jax.numpy as jnp
from jax import lax
from jax.experimental import pallas as pl
from jax.experimental.pallas import tpu as pltpu
from jax.experimental.pallas import tpu_sc as plsc

N_NODES = 10000
D_IN = 128
EMB1 = 64
EMB2 = 16
E = 320000

NC, NS = 2, 16
NW = NC * NS
CHUNK = 128
K = 80
EPW = CHUNK * K
E_PAD = EPW * NW
N_PAD = 10240
ROWS_PER_SUB = N_PAD // NS
PAD_IDX = N_PAD - 1
DEG_W = 16
GROUP = 8
NGRP = K // GROUP
AGRP = 2
ANGRP = K // AGRP

R_BLK = 1024

_mesh = plsc.VectorSubcoreMesh(core_axis_name="c", subcore_axis_name="s",
                               num_cores=NC, num_subcores=NS)
_sc_params = pltpu.CompilerParams(use_tc_tiling_on_sc=False)


@functools.partial(
    pl.kernel,
    out_type=jax.ShapeDtypeStruct((NC * N_PAD, DEG_W), jnp.float32),
    mesh=_mesh,
    scratch_types=[
        *[pltpu.VMEM((CHUNK,), jnp.int32) for _ in range(GROUP)],
        pltpu.VMEM((CHUNK, DEG_W), jnp.float32),
        pltpu.VMEM_SHARED((N_PAD, DEG_W), jnp.float32),
        pltpu.SemaphoreType.DMA,
        pltpu.SemaphoreType.DMA,
    ],
    compiler_params=_sc_params,
)
def _deg_kernel(dst_hbm, ones_hbm, zeros_hbm, out_hbm,
                d0, d1, d2, d3, d4, d5, d6, d7, ones_v, deg_sh, isem, ssem):
    c = lax.axis_index("c")
    s = lax.axis_index("s")
    w = c * NS + s
    didx = [d0, d1, d2, d3, d4, d5, d6, d7]
    pltpu.sync_copy(ones_hbm, ones_v)
    base = s * ROWS_PER_SUB
    pltpu.sync_copy(zeros_hbm, deg_sh.at[pl.ds(base, ROWS_PER_SUB)])
    plsc.subcore_barrier()

    @pl.loop(0, NGRP)
    def _(g):
        e0 = w * EPW + g * GROUP * CHUNK
        loads = [
            pltpu.async_copy(dst_hbm.at[pl.ds(e0 + b * CHUNK, CHUNK)],
                             didx[b], isem)
            for b in range(GROUP)
        ]
        for dsc in loads:
            dsc.wait()
        for b in range(GROUP):
            pltpu.async_copy(ones_v, deg_sh.at[didx[b]], ssem,
                             add=True).wait()

    plsc.subcore_barrier()
    pltpu.sync_copy(deg_sh.at[pl.ds(base, ROWS_PER_SUB)],
                    out_hbm.at[pl.ds(c * N_PAD + base, ROWS_PER_SUB)])


def _make_agg(D):
    @functools.partial(
        pl.kernel,
        out_type=jax.ShapeDtypeStruct((NC * N_PAD, D), jnp.float32),
        mesh=_mesh,
        scratch_types=[
            pltpu.VMEM((K, CHUNK), jnp.int32),
            *[pltpu.VMEM((CHUNK,), jnp.int32) for _ in range(2 * AGRP)],
            pltpu.VMEM((AGRP, CHUNK, D), jnp.float32),
            pltpu.VMEM((AGRP, CHUNK, D), jnp.float32),
            pltpu.VMEM_SHARED((N_PAD, D), jnp.float32),
            pltpu.VMEM_SHARED((N_PAD, D), jnp.float32),
            pltpu.SemaphoreType.DMA,
            pltpu.SemaphoreType.DMA,
            pltpu.SemaphoreType.DMA,
            pltpu.SemaphoreType.DMA,
            pltpu.SemaphoreType.DMA,
            pltpu.SemaphoreType.DMA,
        ],
        compiler_params=_sc_params,
    )
    def agg(h_hbm, src_hbm, dst_hbm, zeros_hbm, out_hbm,
            src_all, d0, d1, d2, d3,
            rows_a, rows_b, h_sh, agg_sh, gsa, gsb, ssa, ssb, isa, isb):
        c = lax.axis_index("c")
        s = lax.axis_index("s")
        w = c * NS + s
        banks = [[d0, d1], [d2, d3]]
        rows = [rows_a, rows_b]
        gsems = [gsa, gsb]
        ssems = [ssa, ssb]
        isems = [isa, isb]
        base = s * ROWS_PER_SUB
        pltpu.sync_copy(src_hbm.at[w], src_all)
        pltpu.sync_copy(zeros_hbm, agg_sh.at[pl.ds(base, ROWS_PER_SUB)])
        pltpu.sync_copy(h_hbm.at[pl.ds(base, ROWS_PER_SUB)],
                        h_sh.at[pl.ds(base, ROWS_PER_SUB)])
        plsc.subcore_barrier()

        def gathers(g, p):
            return [
                pltpu.async_copy(h_sh.at[src_all.at[g * AGRP + b]],
                                 rows[p].at[b], gsems[p])
                for b in range(AGRP)
            ]

        def loads(g, p):
            e0 = w * EPW + g * AGRP * CHUNK
            return [
                pltpu.async_copy(dst_hbm.at[pl.ds(e0 + b * CHUNK, CHUNK)],
                                 banks[p][b], isems[p])
                for b in range(AGRP)
            ]

        pend_g = gathers(0, 0)
        pend_i = loads(0, 0)
        pend_s = None
        for g in range(ANGRP):
            p = g % 2
            for dsc in pend_g:
                dsc.wait()
            for dsc in pend_i:
                dsc.wait()
            if pend_s is not None:
                for dsc in pend_s:
                    dsc.wait()
            if g + 1 < ANGRP:
                pend_g = gathers(g + 1, 1 - p)
                pend_i = loads(g + 1, 1 - p)
            pend_s = [
                pltpu.async_copy(rows[p].at[b], agg_sh.at[banks[p][b]],
                                 ssems[p], add=True)
                for b in range(AGRP)
            ]
        for dsc in pend_s:
            dsc.wait()

        plsc.subcore_barrier()
        pltpu.sync_copy(agg_sh.at[pl.ds(base, ROWS_PER_SUB)],
                        out_hbm.at[pl.ds(c * N_PAD + base, ROWS_PER_SUB)])

    return agg


_agg64 = _make_agg(EMB1)
_agg16 = _make_agg(EMB2)


def _mm_body(x_ref, w_ref, o_ref):
    o_ref[...] = jnp.dot(x_ref[...], w_ref[...],
                         preferred_element_type=jnp.float32,
                         precision=lax.Precision.HIGHEST)


_h1_call = pl.pallas_call(
    _mm_body,
    grid=(N_PAD // R_BLK,),
    in_specs=[pl.BlockSpec((R_BLK, D_IN), lambda i: (i, 0)),
              pl.BlockSpec((D_IN, EMB1), lambda i: (0, 0))],
    out_specs=pl.BlockSpec((R_BLK, EMB1), lambda i: (i, 0)),
    out_shape=jax.ShapeDtypeStruct((N_PAD, EMB1), jnp.float32),
)


def _dis(dp_ref):
    deg = dp_ref[0, :] + dp_ref[1, :] + 1.0
    return lax.rsqrt(deg)


def _scale_body(dp_ref, h_ref, o_ref):
    o_ref[...] = h_ref[...] * _dis(dp_ref)[:, None]


_scale_call = pl.pallas_call(
    _scale_body,
    grid=(N_PAD // R_BLK,),
    in_specs=[pl.BlockSpec((NC, R_BLK), lambda i: (0, i)),
              pl.BlockSpec((R_BLK, EMB1), lambda i: (i, 0))],
    out_specs=pl.BlockSpec((R_BLK, EMB1), lambda i: (i, 0)),
    out_shape=jax.ShapeDtypeStruct((N_PAD, EMB1), jnp.float32),
)


def _layer1_body(p_ref, h_ref, dp_ref, b1_ref, w2_ref, o_ref):
    dis = _dis(dp_ref)
    agg = p_ref[0] + p_ref[1] + h_ref[...]
    z = jnp.maximum(agg * dis[:, None] + b1_ref[0, :][None, :], 0.0)
    o_ref[...] = jnp.dot(z * dis[:, None], w2_ref[...],
                         preferred_element_type=jnp.float32,
                         precision=lax.Precision.HIGHEST)


_layer1_call = pl.pallas_call(
    _layer1_body,
    grid=(N_PAD // R_BLK,),
    in_specs=[pl.BlockSpec((NC, R_BLK, EMB1), lambda i: (0, i, 0)),
              pl.BlockSpec((R_BLK, EMB1), lambda i: (i, 0)),
              pl.BlockSpec((NC, R_BLK), lambda i: (0, i)),
              pl.BlockSpec((1, EMB1), lambda i: (0, 0)),
              pl.BlockSpec((EMB1, EMB2), lambda i: (0, 0))],
    out_specs=pl.BlockSpec((R_BLK, EMB2), lambda i: (i, 0)),
    out_shape=jax.ShapeDtypeStruct((N_PAD, EMB2), jnp.float32),
)


def _layer2_body(q_ref, h2_ref, dp_ref, b2_ref, o_ref):
    dis = _dis(dp_ref)
    agg = q_ref[0] + q_ref[1] + h2_ref[...]
    o_ref[...] = jnp.maximum(agg * dis[:, None] + b2_ref[0, :][None, :], 0.0)


_layer2_call = pl.pallas_call(
    _layer2_body,
    grid=(N_PAD // R_BLK,),
    in_specs=[pl.BlockSpec((NC, R_BLK, EMB2), lambda i: (0, i, 0)),
              pl.BlockSpec((R_BLK, EMB2), lambda i: (i, 0)),
              pl.BlockSpec((NC, R_BLK), lambda i: (0, i)),
              pl.BlockSpec((1, EMB2), lambda i: (0, 0))],
    out_specs=pl.BlockSpec((R_BLK, EMB2), lambda i: (i, 0)),
    out_shape=jax.ShapeDtypeStruct((N_PAD, EMB2), jnp.float32),
)


def kernel(x, edge_index, W1, b1, W2, b2):
    x = x.astype(jnp.float32)
    src = edge_index[0].astype(jnp.int32)
    dst = edge_index[1].astype(jnp.int32)

    xp = jnp.zeros((N_PAD, D_IN), jnp.float32).at[:N_NODES].set(x)
    pad = jnp.full((E_PAD - E,), PAD_IDX, jnp.int32)
    src_r = jnp.concatenate([src, pad]).reshape(NW, K, CHUNK)
    dst_f = jnp.concatenate([dst, pad])

    ones_rows = jnp.ones((CHUNK, DEG_W), jnp.float32)
    zeros16 = jnp.zeros((ROWS_PER_SUB, DEG_W), jnp.float32)
    zeros64 = jnp.zeros((ROWS_PER_SUB, EMB1), jnp.float32)

    degp = _deg_kernel(dst_f, ones_rows, zeros16)
    dp = degp.reshape(NC, N_PAD, DEG_W)[:, :, 0]

    h1 = _h1_call(xp, W1)
    hs1 = _scale_call(dp, h1)
    p = _agg64(hs1, src_r, dst_f, zeros64).reshape(NC, N_PAD, EMB1)
    hs2 = _layer1_call(p, hs1, dp, b1.reshape(1, EMB1), W2)
    q = _agg16(hs2, src_r, dst_f, zeros16).reshape(NC, N_PAD, EMB2)
    out = _layer2_call(q, hs2, dp, b2.reshape(1, EMB2))
    return out[:N_NODES]
